# Initial kernel scaffold; baseline (speedup 1.0000x reference)
import jax
import jax.numpy as jnp
from jax.experimental import pallas as pl


def kernel(x, edge_index, edge_attr, edge_weight, batch, params):
    raise NotImplementedError("write your pallas kernel here")


# speedup vs baseline: 2.1639x; 2.1639x over previous
import functools

import jax
import jax.numpy as jnp
from jax import lax
from jax.experimental import pallas as pl
from jax.experimental.pallas import tpu as pltpu
from jax.experimental.pallas import tpu_sc as plsc

N = 10000
E = 320000
D = 128
DE = 16
G = 64

NC = 2
NS = 16
C = 128
E_PER_SC = E // NC
CHUNKS_PER_SC = E_PER_SC // C
ROWS_PER_TILE = 624
ROWS_REM = N - NS * ROWS_PER_TILE
ZROWS = 8


BE = 2000


def _edge_mlp_body(ea_ref, w1_ref, b1_ref, w2_ref, b2_ref, out_ref):
    t = jnp.maximum(
        jnp.dot(ea_ref[...], w1_ref[...], preferred_element_type=jnp.float32)
        + b1_ref[...], 0.0)
    out_ref[...] = (
        jnp.dot(t, w2_ref[...], preferred_element_type=jnp.float32)
        + b2_ref[...])


def _edge_mlp(edge_attr, w1, b1, w2, b2):
    grid = (E // BE,)
    return pl.pallas_call(
        _edge_mlp_body,
        grid=grid,
        in_specs=[
            pl.BlockSpec((BE, DE), lambda i: (i, 0)),
            pl.BlockSpec((DE, D), lambda i: (0, 0)),
            pl.BlockSpec((1, D), lambda i: (0, 0)),
            pl.BlockSpec((D, D), lambda i: (0, 0)),
            pl.BlockSpec((1, D), lambda i: (0, 0)),
        ],
        out_specs=pl.BlockSpec((BE, D), lambda i: (i, 0)),
        out_shape=jax.ShapeDtypeStruct((E, D), jnp.float32),
    )(edge_attr, w1, b1, w2, b2)


def _ew_expand_body(ew_ref, out_ref):
    out_ref[...] = jnp.broadcast_to(ew_ref[...], (BE, 16))


def _ew_expand(ew2d):
    return pl.pallas_call(
        _ew_expand_body,
        grid=(E // BE,),
        in_specs=[pl.BlockSpec((BE, 1), lambda i: (i, 0))],
        out_specs=pl.BlockSpec((BE, 16), lambda i: (i, 0)),
        out_shape=jax.ShapeDtypeStruct((E, 16), jnp.float32),
    )(ew2d)


def _msg_agg_body(h_hbm, ee_hbm, src_hbm, dst_hbm, ew_hbm, out_hbm,
                  src_v, dst_v, ew_v, hrows_v, ee_v, zbuf_v, agg_sh,
                  sem):
    cid = lax.axis_index("c")
    sid = lax.axis_index("s")

    def zero_row(i, _):
        for d8 in range(D // 16):
            zbuf_v[i, pl.ds(d8 * 16, 16)] = jnp.zeros((16,), jnp.float32)
        return 0
    lax.fori_loop(0, ZROWS, zero_row, 0)
    rows_start = sid * ROWS_PER_TILE

    def zfill(k, _):
        pltpu.sync_copy(zbuf_v, agg_sh.at[pl.ds(rows_start + k * ZROWS,
                                                ZROWS)])
        return 0
    lax.fori_loop(0, ROWS_PER_TILE // ZROWS, zfill, 0)

    @pl.when(sid == NS - 1)
    def _():
        for k in range(ROWS_REM // ZROWS):
            pltpu.sync_copy(
                zbuf_v,
                agg_sh.at[pl.ds(NS * ROWS_PER_TILE + k * ZROWS, ZROWS)])
    plsc.subcore_barrier()

    n_g = (CHUNKS_PER_SC - sid + NS - 1) // NS
    e_base = cid * E_PER_SC

    def chunk_body(j, _):
        chunk = sid + j * NS
        off = e_base + chunk * C
        pltpu.sync_copy(src_hbm.at[pl.ds(off, C)], src_v)
        pltpu.sync_copy(dst_hbm.at[pl.ds(off, C)], dst_v)
        pltpu.sync_copy(ew_hbm.at[pl.ds(off, C)], ew_v)
        pltpu.sync_copy(ee_hbm.at[pl.ds(off, C)], ee_v)
        pltpu.async_copy(h_hbm.at[src_v], hrows_v, sem).wait()

        def edge_body(c, _):
            w = ew_v[c, pl.ds(0, 16)]
            for d8 in range(D // 16):
                sl = pl.ds(d8 * 16, 16)
                hrows_v[c, sl] = jnp.maximum(hrows_v[c, sl] + ee_v[c, sl],
                                             0.0) * w
            return 0
        lax.fori_loop(0, C, edge_body, 0)

        pltpu.sync_copy(hrows_v, agg_sh.at[dst_v], add=True)
        return 0
    lax.fori_loop(0, n_g, chunk_body, 0)

    plsc.subcore_barrier()
    pltpu.sync_copy(agg_sh.at[pl.ds(rows_start, ROWS_PER_TILE)],
                    out_hbm.at[cid, pl.ds(rows_start, ROWS_PER_TILE)])

    @pl.when(sid == NS - 1)
    def _():
        pltpu.sync_copy(agg_sh.at[pl.ds(NS * ROWS_PER_TILE, ROWS_REM)],
                        out_hbm.at[cid, pl.ds(NS * ROWS_PER_TILE,
                                              ROWS_REM)])


def _msg_agg(h, ee, src, dst, ew):
    mesh = plsc.VectorSubcoreMesh(core_axis_name="c", subcore_axis_name="s",
                                  num_cores=NC, num_subcores=NS)
    f = pl.kernel(
        _msg_agg_body,
        mesh=mesh,
        out_type=jax.ShapeDtypeStruct((NC, N, D), jnp.float32),
        scratch_types=[
            pltpu.VMEM((C,), jnp.int32),
            pltpu.VMEM((C,), jnp.int32),
            pltpu.VMEM((C, 16), jnp.float32),
            pltpu.VMEM((C, D), jnp.float32),
            pltpu.VMEM((C, D), jnp.float32),
            pltpu.VMEM((ZROWS, D), jnp.float32),
            pltpu.VMEM_SHARED((N, D), jnp.float32),
            pltpu.SemaphoreType.DMA,
        ],
    )
    return f(h, ee, src, dst, ew)


def _node_update_body(h_ref, parts_ref, eps_ref, w1_ref, b1_ref, w2_ref,
                      b2_ref, gamma_ref, beta_ref, out_ref):
    agg = parts_ref[0] + parts_ref[1]
    z = (1.0 + eps_ref[0, 0]) * h_ref[...] + agg
    z = jnp.maximum(
        jnp.dot(z, w1_ref[...], preferred_element_type=jnp.float32)
        + b1_ref[...], 0.0)
    z = (jnp.dot(z, w2_ref[...], preferred_element_type=jnp.float32)
         + b2_ref[...])
    mu = jnp.mean(z, axis=0, keepdims=True)
    var = jnp.mean((z - mu) ** 2, axis=0, keepdims=True)
    z = (z - mu) / jnp.sqrt(var + 1e-5) * gamma_ref[...] + beta_ref[...]
    out_ref[...] = jnp.maximum(z, 0.0)


def _node_update(h, parts, eps, w1, b1, w2, b2, gamma, beta):
    return pl.pallas_call(
        _node_update_body,
        out_shape=jax.ShapeDtypeStruct((N, D), jnp.float32),
    )(h, parts, eps, w1, b1, w2, b2, gamma, beta)


def _pool_head_body(h_ref, batch_ref, w1_ref, b1_ref, w2_ref, b2_ref,
                    w3_ref, b3_ref, w4_ref, b4_ref, out_ref):
    gids = lax.broadcasted_iota(jnp.int32, (N, G), 1)
    oh = (batch_ref[...] == gids).astype(jnp.float32)
    cnt = jnp.sum(oh, axis=0, keepdims=True)
    pooled = jax.lax.dot_general(
        oh, h_ref[...], (((0,), (0,)), ((), ())),
        preferred_element_type=jnp.float32)
    pooled = pooled / jnp.maximum(cnt, 1.0).reshape(G, 1)
    g = jnp.maximum(
        jnp.dot(pooled, w1_ref[...], preferred_element_type=jnp.float32)
        + b1_ref[...], 0.0)
    g = jnp.maximum(
        jnp.dot(g, w2_ref[...], preferred_element_type=jnp.float32)
        + b2_ref[...], 0.0)
    g = jnp.maximum(
        jnp.dot(g, w3_ref[...], preferred_element_type=jnp.float32)
        + b3_ref[...], 0.0)
    out_ref[...] = (
        jnp.dot(g, w4_ref[...], preferred_element_type=jnp.float32)
        + b4_ref[...])


def _pool_head(h, batch2d, p):
    return pl.pallas_call(
        _pool_head_body,
        out_shape=jax.ShapeDtypeStruct((G, 1), jnp.float32),
    )(h, batch2d,
      p["fcW_1"], p["fcb_1"].reshape(1, D),
      p["fcW_2"], p["fcb_2"].reshape(1, D),
      p["fcW_3"], p["fcb_3"].reshape(1, D),
      p["fcW_4"], p["fcb_4"].reshape(1, 1))


def kernel(x, edge_index, edge_attr, edge_weight, batch, params):
    src = edge_index[0]
    dst = edge_index[1]
    L = 3
    ee = [
        _edge_mlp(edge_attr,
                  params[f"Wb1_{l}"], params[f"bb1_{l}"].reshape(1, D),
                  params[f"Wb2_{l}"], params[f"bb2_{l}"].reshape(1, D))
        for l in range(L)
    ]
    ewb = _ew_expand(edge_weight.reshape(E, 1))
    h = x
    for l in range(L):
        parts = _msg_agg(h, ee[l], src, dst, ewb)
        h = _node_update(
            h, parts, params[f"eps_{l}"].reshape(1, 1),
            params[f"Wm1_{l}"], params[f"bm1_{l}"].reshape(1, D),
            params[f"Wm2_{l}"], params[f"bm2_{l}"].reshape(1, D),
            params[f"gamma_{l}"].reshape(1, D),
            params[f"beta_{l}"].reshape(1, D))
    out = _pool_head(h, batch.reshape(N, 1), params)
    return out.reshape(-1)


# speedup vs baseline: 2.7791x; 1.2843x over previous
import functools

import jax
import jax.numpy as jnp
from jax import lax
from jax.experimental import pallas as pl
from jax.experimental.pallas import tpu as pltpu
from jax.experimental.pallas import tpu_sc as plsc

N = 10000
E = 320000
D = 128
DE = 16
G = 64

NC = 2
NS = 16
C = 64
E_PER_SC = E // NC
CHUNKS_PER_SC = E_PER_SC // C
ROWS_PER_TILE = 624
ROWS_REM = N - NS * ROWS_PER_TILE
ZROWS = 8


BE = 2000


def _edge_mlp_body(ea_ref, w1_ref, b1_ref, w2_ref, b2_ref, out_ref):
    t = jnp.maximum(
        jnp.dot(ea_ref[...], w1_ref[...], preferred_element_type=jnp.float32)
        + b1_ref[...], 0.0)
    out_ref[...] = (
        jnp.dot(t, w2_ref[...], preferred_element_type=jnp.float32)
        + b2_ref[...])


def _edge_mlp(edge_attr, w1, b1, w2, b2):
    grid = (E // BE,)
    return pl.pallas_call(
        _edge_mlp_body,
        grid=grid,
        in_specs=[
            pl.BlockSpec((BE, DE), lambda i: (i, 0)),
            pl.BlockSpec((DE, D), lambda i: (0, 0)),
            pl.BlockSpec((1, D), lambda i: (0, 0)),
            pl.BlockSpec((D, D), lambda i: (0, 0)),
            pl.BlockSpec((1, D), lambda i: (0, 0)),
        ],
        out_specs=pl.BlockSpec((BE, D), lambda i: (i, 0)),
        out_shape=jax.ShapeDtypeStruct((E, D), jnp.float32),
    )(edge_attr, w1, b1, w2, b2)


def _ew_expand_body(ew_ref, out_ref):
    out_ref[...] = jnp.broadcast_to(ew_ref[...], (BE, 16))


def _ew_expand(ew2d):
    return pl.pallas_call(
        _ew_expand_body,
        grid=(E // BE,),
        in_specs=[pl.BlockSpec((BE, 1), lambda i: (i, 0))],
        out_specs=pl.BlockSpec((BE, 16), lambda i: (i, 0)),
        out_shape=jax.ShapeDtypeStruct((E, 16), jnp.float32),
    )(ew2d)


def _msg_agg_body(h_hbm, ee_hbm, src_hbm, dst_hbm, ew_hbm, out_hbm,
                  src0, dst0, ew0, hr0, eev0,
                  src1, dst1, ew1, hr1, eev1,
                  zbuf_v, agg_sh, sem_i0, sem_i1, sem_g0, sem_g1):
    bufs = ((src0, dst0, ew0, hr0, eev0, sem_i0, sem_g0),
            (src1, dst1, ew1, hr1, eev1, sem_i1, sem_g1))
    cid = lax.axis_index("c")
    sid = lax.axis_index("s")

    def zero_row(i, _):
        for d8 in range(D // 16):
            zbuf_v[i, pl.ds(d8 * 16, 16)] = jnp.zeros((16,), jnp.float32)
        return 0
    lax.fori_loop(0, ZROWS, zero_row, 0)
    rows_start = sid * ROWS_PER_TILE

    def zfill(k, _):
        pltpu.sync_copy(zbuf_v, agg_sh.at[pl.ds(rows_start + k * ZROWS,
                                                ZROWS)])
        return 0
    lax.fori_loop(0, ROWS_PER_TILE // ZROWS, zfill, 0)

    @pl.when(sid == NS - 1)
    def _():
        for k in range(ROWS_REM // ZROWS):
            pltpu.sync_copy(
                zbuf_v,
                agg_sh.at[pl.ds(NS * ROWS_PER_TILE + k * ZROWS, ZROWS)])
    plsc.subcore_barrier()

    n_g = (CHUNKS_PER_SC - sid + NS - 1) // NS
    e_base = cid * E_PER_SC

    def issue_idx(j, par):
        src_v, dst_v, ew_v, _, ee_v, sem_i, _ = bufs[par]
        off = e_base + (sid + j * NS) * C
        pltpu.async_copy(src_hbm.at[pl.ds(off, C)], src_v, sem_i)
        pltpu.async_copy(dst_hbm.at[pl.ds(off, C)], dst_v, sem_i)
        pltpu.async_copy(ew_hbm.at[pl.ds(off, C)], ew_v, sem_i)
        pltpu.async_copy(ee_hbm.at[pl.ds(off, C)], ee_v, sem_i)

    def wait_idx(par):
        src_v, dst_v, ew_v, _, ee_v, sem_i, _ = bufs[par]
        pltpu.make_async_copy(src_hbm.at[pl.ds(0, C)], src_v, sem_i).wait()
        pltpu.make_async_copy(dst_hbm.at[pl.ds(0, C)], dst_v, sem_i).wait()
        pltpu.make_async_copy(ew_hbm.at[pl.ds(0, C)], ew_v, sem_i).wait()
        pltpu.make_async_copy(ee_hbm.at[pl.ds(0, C)], ee_v, sem_i).wait()

    def issue_gather(par):
        src_v, _, _, hr_v, _, _, sem_g = bufs[par]
        pltpu.async_copy(h_hbm.at[src_v], hr_v, sem_g)

    def wait_gather(par):
        src_v, _, _, hr_v, _, _, sem_g = bufs[par]
        pltpu.make_async_copy(h_hbm.at[src_v], hr_v, sem_g).wait()

    issue_idx(0, 0)
    wait_idx(0)
    issue_gather(0)
    issue_idx(1, 1)

    def pair_body(jj, _):
        for par in range(2):
            j = jj * 2 + par
            nb = 1 - par

            @pl.when(j < n_g)
            def _():
                wait_gather(par)

                @pl.when(j + 1 < n_g)
                def _():
                    wait_idx(nb)
                    issue_gather(nb)

                src_v, dst_v, ew_v, hr_v, ee_v, _, _ = bufs[par]

                def edge_body(c, _):
                    w = ew_v[c, pl.ds(0, 16)]
                    for d8 in range(D // 16):
                        sl = pl.ds(d8 * 16, 16)
                        hr_v[c, sl] = jnp.maximum(
                            hr_v[c, sl] + ee_v[c, sl], 0.0) * w
                    return 0
                lax.fori_loop(0, C, edge_body, 0)

                pltpu.sync_copy(hr_v, agg_sh.at[dst_v], add=True)

                @pl.when(j + 2 < n_g)
                def _():
                    issue_idx(j + 2, par)
        return 0
    lax.fori_loop(0, (n_g + 1) // 2, pair_body, 0)

    plsc.subcore_barrier()
    pltpu.sync_copy(agg_sh.at[pl.ds(rows_start, ROWS_PER_TILE)],
                    out_hbm.at[cid, pl.ds(rows_start, ROWS_PER_TILE)])

    @pl.when(sid == NS - 1)
    def _():
        pltpu.sync_copy(agg_sh.at[pl.ds(NS * ROWS_PER_TILE, ROWS_REM)],
                        out_hbm.at[cid, pl.ds(NS * ROWS_PER_TILE,
                                              ROWS_REM)])


def _msg_agg(h, ee, src, dst, ew):
    mesh = plsc.VectorSubcoreMesh(core_axis_name="c", subcore_axis_name="s",
                                  num_cores=NC, num_subcores=NS)
    f = pl.kernel(
        _msg_agg_body,
        mesh=mesh,
        out_type=jax.ShapeDtypeStruct((NC, N, D), jnp.float32),
        scratch_types=(
            [pltpu.VMEM((C,), jnp.int32),
             pltpu.VMEM((C,), jnp.int32),
             pltpu.VMEM((C, 16), jnp.float32),
             pltpu.VMEM((C, D), jnp.float32),
             pltpu.VMEM((C, D), jnp.float32)] * 2
            + [pltpu.VMEM((ZROWS, D), jnp.float32),
               pltpu.VMEM_SHARED((N, D), jnp.float32),
               pltpu.SemaphoreType.DMA,
               pltpu.SemaphoreType.DMA,
               pltpu.SemaphoreType.DMA,
               pltpu.SemaphoreType.DMA]),
    )
    return f(h, ee, src, dst, ew)


def _node_update_body(h_ref, parts_ref, eps_ref, w1_ref, b1_ref, w2_ref,
                      b2_ref, gamma_ref, beta_ref, out_ref):
    agg = parts_ref[0] + parts_ref[1]
    z = (1.0 + eps_ref[0, 0]) * h_ref[...] + agg
    z = jnp.maximum(
        jnp.dot(z, w1_ref[...], preferred_element_type=jnp.float32)
        + b1_ref[...], 0.0)
    z = (jnp.dot(z, w2_ref[...], preferred_element_type=jnp.float32)
         + b2_ref[...])
    mu = jnp.mean(z, axis=0, keepdims=True)
    var = jnp.mean((z - mu) ** 2, axis=0, keepdims=True)
    z = (z - mu) / jnp.sqrt(var + 1e-5) * gamma_ref[...] + beta_ref[...]
    out_ref[...] = jnp.maximum(z, 0.0)


def _node_update(h, parts, eps, w1, b1, w2, b2, gamma, beta):
    return pl.pallas_call(
        _node_update_body,
        out_shape=jax.ShapeDtypeStruct((N, D), jnp.float32),
    )(h, parts, eps, w1, b1, w2, b2, gamma, beta)


def _pool_head_body(h_ref, batch_ref, w1_ref, b1_ref, w2_ref, b2_ref,
                    w3_ref, b3_ref, w4_ref, b4_ref, out_ref):
    gids = lax.broadcasted_iota(jnp.int32, (N, G), 1)
    oh = (batch_ref[...] == gids).astype(jnp.float32)
    cnt = jnp.sum(oh, axis=0, keepdims=True)
    pooled = jax.lax.dot_general(
        oh, h_ref[...], (((0,), (0,)), ((), ())),
        preferred_element_type=jnp.float32)
    pooled = pooled / jnp.maximum(cnt, 1.0).reshape(G, 1)
    g = jnp.maximum(
        jnp.dot(pooled, w1_ref[...], preferred_element_type=jnp.float32)
        + b1_ref[...], 0.0)
    g = jnp.maximum(
        jnp.dot(g, w2_ref[...], preferred_element_type=jnp.float32)
        + b2_ref[...], 0.0)
    g = jnp.maximum(
        jnp.dot(g, w3_ref[...], preferred_element_type=jnp.float32)
        + b3_ref[...], 0.0)
    out_ref[...] = (
        jnp.dot(g, w4_ref[...], preferred_element_type=jnp.float32)
        + b4_ref[...])


def _pool_head(h, batch2d, p):
    return pl.pallas_call(
        _pool_head_body,
        out_shape=jax.ShapeDtypeStruct((G, 1), jnp.float32),
    )(h, batch2d,
      p["fcW_1"], p["fcb_1"].reshape(1, D),
      p["fcW_2"], p["fcb_2"].reshape(1, D),
      p["fcW_3"], p["fcb_3"].reshape(1, D),
      p["fcW_4"], p["fcb_4"].reshape(1, 1))


def kernel(x, edge_index, edge_attr, edge_weight, batch, params):
    src = edge_index[0]
    dst = edge_index[1]
    L = 3
    ee = [
        _edge_mlp(edge_attr,
                  params[f"Wb1_{l}"], params[f"bb1_{l}"].reshape(1, D),
                  params[f"Wb2_{l}"], params[f"bb2_{l}"].reshape(1, D))
        for l in range(L)
    ]
    ewb = _ew_expand(edge_weight.reshape(E, 1))
    h = x
    for l in range(L):
        parts = _msg_agg(h, ee[l], src, dst, ewb)
        h = _node_update(
            h, parts, params[f"eps_{l}"].reshape(1, 1),
            params[f"Wm1_{l}"], params[f"bm1_{l}"].reshape(1, D),
            params[f"Wm2_{l}"], params[f"bm2_{l}"].reshape(1, D),
            params[f"gamma_{l}"].reshape(1, D),
            params[f"beta_{l}"].reshape(1, D))
    out = _pool_head(h, batch.reshape(N, 1), params)
    return out.reshape(-1)


# speedup vs baseline: 3.1634x; 1.1383x over previous
import functools

import jax
import jax.numpy as jnp
from jax import lax
from jax.experimental import pallas as pl
from jax.experimental.pallas import tpu as pltpu
from jax.experimental.pallas import tpu_sc as plsc

N = 10000
E = 320000
D = 128
DE = 16
G = 64

NC = 2
NS = 16
C = 64
E_PER_SC = E // NC
CHUNKS_PER_SC = E_PER_SC // C
ROWS_PER_TILE = 624
ROWS_REM = N - NS * ROWS_PER_TILE
ZROWS = 8


BE = 2000


def _edge_mlp3_body(ea_ref, ew_ref, w1_ref, b1_ref, w2_ref, b2_ref,
                    o0_ref, o1_ref, o2_ref, ewb_ref):
    ea = ea_ref[...]
    for l, o_ref in enumerate((o0_ref, o1_ref, o2_ref)):
        t = jnp.maximum(
            jnp.dot(ea, w1_ref[l], preferred_element_type=jnp.float32)
            + b1_ref[l], 0.0)
        o_ref[...] = (
            jnp.dot(t, w2_ref[l], preferred_element_type=jnp.float32)
            + b2_ref[l])
    ewb_ref[...] = jnp.broadcast_to(ew_ref[...], (BE, 16))


def _edge_mlp3(edge_attr, ew2d, w1s, b1s, w2s, b2s):
    return pl.pallas_call(
        _edge_mlp3_body,
        grid=(E // BE,),
        in_specs=[
            pl.BlockSpec((BE, DE), lambda i: (i, 0)),
            pl.BlockSpec((BE, 1), lambda i: (i, 0)),
            pl.BlockSpec((3, DE, D), lambda i: (0, 0, 0)),
            pl.BlockSpec((3, 1, D), lambda i: (0, 0, 0)),
            pl.BlockSpec((3, D, D), lambda i: (0, 0, 0)),
            pl.BlockSpec((3, 1, D), lambda i: (0, 0, 0)),
        ],
        out_specs=[
            pl.BlockSpec((BE, D), lambda i: (i, 0)),
            pl.BlockSpec((BE, D), lambda i: (i, 0)),
            pl.BlockSpec((BE, D), lambda i: (i, 0)),
            pl.BlockSpec((BE, 16), lambda i: (i, 0)),
        ],
        out_shape=[
            jax.ShapeDtypeStruct((E, D), jnp.float32),
            jax.ShapeDtypeStruct((E, D), jnp.float32),
            jax.ShapeDtypeStruct((E, D), jnp.float32),
            jax.ShapeDtypeStruct((E, 16), jnp.float32),
        ],
    )(edge_attr, ew2d, w1s, b1s, w2s, b2s)


def _msg_agg_body(h_hbm, ee_hbm, src_hbm, dst_hbm, ew_hbm, out_hbm,
                  src0, dst0, ew0, hr0, eev0,
                  src1, dst1, ew1, hr1, eev1,
                  zbuf_v, agg_sh, sem_i0, sem_i1, sem_g0, sem_g1,
                  sem_d0, sem_d1, sem_s0, sem_s1):
    bufs = ((src0, dst0, ew0, hr0, eev0, sem_i0, sem_g0, sem_d0, sem_s0),
            (src1, dst1, ew1, hr1, eev1, sem_i1, sem_g1, sem_d1, sem_s1))
    cid = lax.axis_index("c")
    sid = lax.axis_index("s")

    def zero_row(i, _):
        for d8 in range(D // 16):
            zbuf_v[i, pl.ds(d8 * 16, 16)] = jnp.zeros((16,), jnp.float32)
        return 0
    lax.fori_loop(0, ZROWS, zero_row, 0)
    rows_start = sid * ROWS_PER_TILE

    def zfill(k, _):
        pltpu.sync_copy(zbuf_v, agg_sh.at[pl.ds(rows_start + k * ZROWS,
                                                ZROWS)])
        return 0
    lax.fori_loop(0, ROWS_PER_TILE // ZROWS, zfill, 0)

    @pl.when(sid == NS - 1)
    def _():
        for k in range(ROWS_REM // ZROWS):
            pltpu.sync_copy(
                zbuf_v,
                agg_sh.at[pl.ds(NS * ROWS_PER_TILE + k * ZROWS, ZROWS)])
    plsc.subcore_barrier()

    n_g = (CHUNKS_PER_SC - sid + NS - 1) // NS
    e_base = cid * E_PER_SC

    def issue_idx(j, par):
        src_v, _, ew_v, _, ee_v = bufs[par][:5]
        sem_i = bufs[par][5]
        off = e_base + (sid + j * NS) * C
        pltpu.async_copy(src_hbm.at[pl.ds(off, C)], src_v, sem_i)
        pltpu.async_copy(ew_hbm.at[pl.ds(off, C)], ew_v, sem_i)
        pltpu.async_copy(ee_hbm.at[pl.ds(off, C)], ee_v, sem_i)

    def wait_idx(par):
        src_v, _, ew_v, _, ee_v = bufs[par][:5]
        sem_i = bufs[par][5]
        pltpu.make_async_copy(src_hbm.at[pl.ds(0, C)], src_v, sem_i).wait()
        pltpu.make_async_copy(ew_hbm.at[pl.ds(0, C)], ew_v, sem_i).wait()
        pltpu.make_async_copy(ee_hbm.at[pl.ds(0, C)], ee_v, sem_i).wait()

    def issue_dst(j, par):
        dst_v, sem_d = bufs[par][1], bufs[par][7]
        off = e_base + (sid + j * NS) * C
        pltpu.async_copy(dst_hbm.at[pl.ds(off, C)], dst_v, sem_d)

    def wait_dst(par):
        dst_v, sem_d = bufs[par][1], bufs[par][7]
        pltpu.make_async_copy(dst_hbm.at[pl.ds(0, C)], dst_v, sem_d).wait()

    def issue_gather(par):
        src_v, hr_v, sem_g = bufs[par][0], bufs[par][3], bufs[par][6]
        pltpu.async_copy(h_hbm.at[src_v], hr_v, sem_g)

    def wait_gather(par):
        src_v, hr_v, sem_g = bufs[par][0], bufs[par][3], bufs[par][6]
        pltpu.make_async_copy(h_hbm.at[src_v], hr_v, sem_g).wait()

    def issue_scatter(par):
        dst_v, hr_v, sem_s = bufs[par][1], bufs[par][3], bufs[par][8]
        pltpu.async_copy(hr_v, agg_sh.at[dst_v], sem_s, add=True)

    def wait_scatter(par):
        dst_v, hr_v, sem_s = bufs[par][1], bufs[par][3], bufs[par][8]
        pltpu.make_async_copy(hr_v, agg_sh.at[dst_v], sem_s).wait()

    issue_idx(0, 0)
    issue_dst(0, 0)
    wait_idx(0)
    issue_gather(0)
    issue_idx(1, 1)

    def pair_body(jj, _):
        for par in range(2):
            j = jj * 2 + par
            nb = 1 - par

            @pl.when(j < n_g)
            def _():
                wait_gather(par)

                @pl.when(j + 1 < n_g)
                def _():
                    wait_idx(nb)

                    @pl.when(j >= 1)
                    def _():
                        wait_scatter(nb)
                    issue_gather(nb)
                    issue_dst(j + 1, nb)

                src_v, dst_v, ew_v, hr_v, ee_v = bufs[par][:5]

                def edge_body(c, _):
                    w = ew_v[c, pl.ds(0, 16)]
                    for d8 in range(D // 16):
                        sl = pl.ds(d8 * 16, 16)
                        hr_v[c, sl] = jnp.maximum(
                            hr_v[c, sl] + ee_v[c, sl], 0.0) * w
                    return 0
                lax.fori_loop(0, C, edge_body, 0)

                wait_dst(par)
                issue_scatter(par)

                @pl.when(j + 2 < n_g)
                def _():
                    issue_idx(j + 2, par)
        return 0
    lax.fori_loop(0, (n_g + 1) // 2, pair_body, 0)
    wait_scatter(0)
    wait_scatter(1)

    plsc.subcore_barrier()
    pltpu.sync_copy(agg_sh.at[pl.ds(rows_start, ROWS_PER_TILE)],
                    out_hbm.at[cid, pl.ds(rows_start, ROWS_PER_TILE)])

    @pl.when(sid == NS - 1)
    def _():
        pltpu.sync_copy(agg_sh.at[pl.ds(NS * ROWS_PER_TILE, ROWS_REM)],
                        out_hbm.at[cid, pl.ds(NS * ROWS_PER_TILE,
                                              ROWS_REM)])


def _msg_agg(h, ee, src, dst, ew):
    mesh = plsc.VectorSubcoreMesh(core_axis_name="c", subcore_axis_name="s",
                                  num_cores=NC, num_subcores=NS)
    f = pl.kernel(
        _msg_agg_body,
        mesh=mesh,
        out_type=jax.ShapeDtypeStruct((NC, N, D), jnp.float32),
        scratch_types=(
            [pltpu.VMEM((C,), jnp.int32),
             pltpu.VMEM((C,), jnp.int32),
             pltpu.VMEM((C, 16), jnp.float32),
             pltpu.VMEM((C, D), jnp.float32),
             pltpu.VMEM((C, D), jnp.float32)] * 2
            + [pltpu.VMEM((ZROWS, D), jnp.float32),
               pltpu.VMEM_SHARED((N, D), jnp.float32)]
            + [pltpu.SemaphoreType.DMA] * 8),
    )
    return f(h, ee, src, dst, ew)


def _node_update_body(h_ref, parts_ref, eps_ref, w1_ref, b1_ref, w2_ref,
                      b2_ref, gamma_ref, beta_ref, out_ref):
    agg = parts_ref[0] + parts_ref[1]
    z = (1.0 + eps_ref[0, 0]) * h_ref[...] + agg
    z = jnp.maximum(
        jnp.dot(z, w1_ref[...], preferred_element_type=jnp.float32)
        + b1_ref[...], 0.0)
    z = (jnp.dot(z, w2_ref[...], preferred_element_type=jnp.float32)
         + b2_ref[...])
    mu = jnp.mean(z, axis=0, keepdims=True)
    var = jnp.mean((z - mu) ** 2, axis=0, keepdims=True)
    z = (z - mu) / jnp.sqrt(var + 1e-5) * gamma_ref[...] + beta_ref[...]
    out_ref[...] = jnp.maximum(z, 0.0)


def _node_update(h, parts, eps, w1, b1, w2, b2, gamma, beta):
    return pl.pallas_call(
        _node_update_body,
        out_shape=jax.ShapeDtypeStruct((N, D), jnp.float32),
    )(h, parts, eps, w1, b1, w2, b2, gamma, beta)


def _pool_head_body(h_ref, batch_ref, w1_ref, b1_ref, w2_ref, b2_ref,
                    w3_ref, b3_ref, w4_ref, b4_ref, out_ref):
    gids = lax.broadcasted_iota(jnp.int32, (N, G), 1)
    oh = (batch_ref[...] == gids).astype(jnp.float32)
    cnt = jnp.sum(oh, axis=0, keepdims=True)
    pooled = jax.lax.dot_general(
        oh, h_ref[...], (((0,), (0,)), ((), ())),
        preferred_element_type=jnp.float32)
    pooled = pooled / jnp.maximum(cnt, 1.0).reshape(G, 1)
    g = jnp.maximum(
        jnp.dot(pooled, w1_ref[...], preferred_element_type=jnp.float32)
        + b1_ref[...], 0.0)
    g = jnp.maximum(
        jnp.dot(g, w2_ref[...], preferred_element_type=jnp.float32)
        + b2_ref[...], 0.0)
    g = jnp.maximum(
        jnp.dot(g, w3_ref[...], preferred_element_type=jnp.float32)
        + b3_ref[...], 0.0)
    out_ref[...] = (
        jnp.dot(g, w4_ref[...], preferred_element_type=jnp.float32)
        + b4_ref[...])


def _pool_head(h, batch2d, p):
    return pl.pallas_call(
        _pool_head_body,
        out_shape=jax.ShapeDtypeStruct((G, 1), jnp.float32),
    )(h, batch2d,
      p["fcW_1"], p["fcb_1"].reshape(1, D),
      p["fcW_2"], p["fcb_2"].reshape(1, D),
      p["fcW_3"], p["fcb_3"].reshape(1, D),
      p["fcW_4"], p["fcb_4"].reshape(1, 1))


def kernel(x, edge_index, edge_attr, edge_weight, batch, params):
    src = edge_index[0]
    dst = edge_index[1]
    L = 3
    w1s = jnp.stack([params[f"Wb1_{l}"] for l in range(L)])
    b1s = jnp.stack([params[f"bb1_{l}"].reshape(1, D) for l in range(L)])
    w2s = jnp.stack([params[f"Wb2_{l}"] for l in range(L)])
    b2s = jnp.stack([params[f"bb2_{l}"].reshape(1, D) for l in range(L)])
    ee0, ee1, ee2, ewb = _edge_mlp3(edge_attr, edge_weight.reshape(E, 1),
                                    w1s, b1s, w2s, b2s)
    ee = (ee0, ee1, ee2)
    h = x
    for l in range(L):
        parts = _msg_agg(h, ee[l], src, dst, ewb)
        h = _node_update(
            h, parts, params[f"eps_{l}"].reshape(1, 1),
            params[f"Wm1_{l}"], params[f"bm1_{l}"].reshape(1, D),
            params[f"Wm2_{l}"], params[f"bm2_{l}"].reshape(1, D),
            params[f"gamma_{l}"].reshape(1, D),
            params[f"beta_{l}"].reshape(1, D))
    out = _pool_head(h, batch.reshape(N, 1), params)
    return out.reshape(-1)


# speedup vs baseline: 3.2311x; 1.0214x over previous
import functools

import jax
import jax.numpy as jnp
from jax import lax
from jax.experimental import pallas as pl
from jax.experimental.pallas import tpu as pltpu
from jax.experimental.pallas import tpu_sc as plsc

N = 10000
E = 320000
D = 128
DE = 16
G = 64

NC = 2
NS = 16
C = 64
E_PER_SC = E // NC
CHUNKS_PER_SC = E_PER_SC // C
ROWS_PER_TILE = 624
ROWS_REM = N - NS * ROWS_PER_TILE
ZROWS = 8


BE = 2000


def _edge_mlp1_body(ea_ref, ew_ref, w1_ref, b1_ref, w2_ref, b2_ref,
                    o0_ref, ewb_ref):
    t = jnp.maximum(
        jnp.dot(ea_ref[...], w1_ref[...],
                preferred_element_type=jnp.float32) + b1_ref[...], 0.0)
    o0_ref[...] = (
        jnp.dot(t, w2_ref[...], preferred_element_type=jnp.float32)
        + b2_ref[...])
    ewb_ref[...] = jnp.broadcast_to(ew_ref[...], (BE, 16))


def _edge_mlp1(edge_attr, ew2d, w1, b1, w2, b2):
    return pl.pallas_call(
        _edge_mlp1_body,
        grid=(E // BE,),
        in_specs=[
            pl.BlockSpec((BE, DE), lambda i: (i, 0)),
            pl.BlockSpec((BE, 1), lambda i: (i, 0)),
            pl.BlockSpec((DE, D), lambda i: (0, 0)),
            pl.BlockSpec((1, D), lambda i: (0, 0)),
            pl.BlockSpec((D, D), lambda i: (0, 0)),
            pl.BlockSpec((1, D), lambda i: (0, 0)),
        ],
        out_specs=[
            pl.BlockSpec((BE, D), lambda i: (i, 0)),
            pl.BlockSpec((BE, 16), lambda i: (i, 0)),
        ],
        out_shape=[
            jax.ShapeDtypeStruct((E, D), jnp.float32),
            jax.ShapeDtypeStruct((E, 16), jnp.float32),
        ],
    )(edge_attr, ew2d, w1, b1, w2, b2)


def _edge_mlp2_body(ea_ref, w1_ref, b1_ref, w2_ref, b2_ref,
                    o1_ref, o2_ref):
    ea = ea_ref[...]
    for l, o_ref in enumerate((o1_ref, o2_ref)):
        t = jnp.maximum(
            jnp.dot(ea, w1_ref[l], preferred_element_type=jnp.float32)
            + b1_ref[l], 0.0)
        o_ref[...] = (
            jnp.dot(t, w2_ref[l], preferred_element_type=jnp.float32)
            + b2_ref[l])


def _edge_mlp2(edge_attr, w1s, b1s, w2s, b2s):
    return pl.pallas_call(
        _edge_mlp2_body,
        grid=(E // BE,),
        in_specs=[
            pl.BlockSpec((BE, DE), lambda i: (i, 0)),
            pl.BlockSpec((2, DE, D), lambda i: (0, 0, 0)),
            pl.BlockSpec((2, 1, D), lambda i: (0, 0, 0)),
            pl.BlockSpec((2, D, D), lambda i: (0, 0, 0)),
            pl.BlockSpec((2, 1, D), lambda i: (0, 0, 0)),
        ],
        out_specs=[
            pl.BlockSpec((BE, D), lambda i: (i, 0)),
            pl.BlockSpec((BE, D), lambda i: (i, 0)),
        ],
        out_shape=[
            jax.ShapeDtypeStruct((E, D), jnp.float32),
            jax.ShapeDtypeStruct((E, D), jnp.float32),
        ],
    )(edge_attr, w1s, b1s, w2s, b2s)


def _msg_agg_body(h_hbm, ee_hbm, src_hbm, dst_hbm, ew_hbm, out_hbm,
                  src0, dst0, ew0, hr0, eev0,
                  src1, dst1, ew1, hr1, eev1,
                  zbuf_v, agg_sh, sem_i0, sem_i1, sem_g0, sem_g1,
                  sem_d0, sem_d1, sem_s0, sem_s1):
    bufs = ((src0, dst0, ew0, hr0, eev0, sem_i0, sem_g0, sem_d0, sem_s0),
            (src1, dst1, ew1, hr1, eev1, sem_i1, sem_g1, sem_d1, sem_s1))
    cid = lax.axis_index("c")
    sid = lax.axis_index("s")

    def zero_row(i, _):
        for d8 in range(D // 16):
            zbuf_v[i, pl.ds(d8 * 16, 16)] = jnp.zeros((16,), jnp.float32)
        return 0
    lax.fori_loop(0, ZROWS, zero_row, 0)
    rows_start = sid * ROWS_PER_TILE

    def zfill(k, _):
        pltpu.sync_copy(zbuf_v, agg_sh.at[pl.ds(rows_start + k * ZROWS,
                                                ZROWS)])
        return 0
    lax.fori_loop(0, ROWS_PER_TILE // ZROWS, zfill, 0)

    @pl.when(sid == NS - 1)
    def _():
        for k in range(ROWS_REM // ZROWS):
            pltpu.sync_copy(
                zbuf_v,
                agg_sh.at[pl.ds(NS * ROWS_PER_TILE + k * ZROWS, ZROWS)])
    plsc.subcore_barrier()

    n_g = (CHUNKS_PER_SC - sid + NS - 1) // NS
    e_base = cid * E_PER_SC

    def issue_idx(j, par):
        src_v, _, ew_v, _, ee_v = bufs[par][:5]
        sem_i = bufs[par][5]
        off = e_base + (sid + j * NS) * C
        pltpu.async_copy(src_hbm.at[pl.ds(off, C)], src_v, sem_i)
        pltpu.async_copy(ew_hbm.at[pl.ds(off, C)], ew_v, sem_i)
        pltpu.async_copy(ee_hbm.at[pl.ds(off, C)], ee_v, sem_i)

    def wait_idx(par):
        src_v, _, ew_v, _, ee_v = bufs[par][:5]
        sem_i = bufs[par][5]
        pltpu.make_async_copy(src_hbm.at[pl.ds(0, C)], src_v, sem_i).wait()
        pltpu.make_async_copy(ew_hbm.at[pl.ds(0, C)], ew_v, sem_i).wait()
        pltpu.make_async_copy(ee_hbm.at[pl.ds(0, C)], ee_v, sem_i).wait()

    def issue_dst(j, par):
        dst_v, sem_d = bufs[par][1], bufs[par][7]
        off = e_base + (sid + j * NS) * C
        pltpu.async_copy(dst_hbm.at[pl.ds(off, C)], dst_v, sem_d)

    def wait_dst(par):
        dst_v, sem_d = bufs[par][1], bufs[par][7]
        pltpu.make_async_copy(dst_hbm.at[pl.ds(0, C)], dst_v, sem_d).wait()

    def issue_gather(par):
        src_v, hr_v, sem_g = bufs[par][0], bufs[par][3], bufs[par][6]
        pltpu.async_copy(h_hbm.at[src_v], hr_v, sem_g)

    def wait_gather(par):
        src_v, hr_v, sem_g = bufs[par][0], bufs[par][3], bufs[par][6]
        pltpu.make_async_copy(h_hbm.at[src_v], hr_v, sem_g).wait()

    def issue_scatter(par):
        dst_v, hr_v, sem_s = bufs[par][1], bufs[par][3], bufs[par][8]
        pltpu.async_copy(hr_v, agg_sh.at[dst_v], sem_s, add=True)

    def wait_scatter(par):
        dst_v, hr_v, sem_s = bufs[par][1], bufs[par][3], bufs[par][8]
        pltpu.make_async_copy(hr_v, agg_sh.at[dst_v], sem_s).wait()

    issue_idx(0, 0)
    issue_dst(0, 0)
    wait_idx(0)
    issue_gather(0)
    issue_idx(1, 1)

    def pair_body(jj, _):
        for par in range(2):
            j = jj * 2 + par
            nb = 1 - par

            @pl.when(j < n_g)
            def _():
                wait_gather(par)

                @pl.when(j + 1 < n_g)
                def _():
                    wait_idx(nb)

                    @pl.when(j >= 1)
                    def _():
                        wait_scatter(nb)
                    issue_gather(nb)
                    issue_dst(j + 1, nb)

                src_v, dst_v, ew_v, hr_v, ee_v = bufs[par][:5]

                def edge_body(c, _):
                    w = ew_v[c, pl.ds(0, 16)]
                    for d8 in range(D // 16):
                        sl = pl.ds(d8 * 16, 16)
                        hr_v[c, sl] = jnp.maximum(
                            hr_v[c, sl] + ee_v[c, sl], 0.0) * w
                    return 0
                lax.fori_loop(0, C, edge_body, 0)

                wait_dst(par)
                issue_scatter(par)

                @pl.when(j + 2 < n_g)
                def _():
                    issue_idx(j + 2, par)
        return 0
    lax.fori_loop(0, (n_g + 1) // 2, pair_body, 0)
    wait_scatter(0)
    wait_scatter(1)

    plsc.subcore_barrier()
    pltpu.sync_copy(agg_sh.at[pl.ds(rows_start, ROWS_PER_TILE)],
                    out_hbm.at[cid, pl.ds(rows_start, ROWS_PER_TILE)])

    @pl.when(sid == NS - 1)
    def _():
        pltpu.sync_copy(agg_sh.at[pl.ds(NS * ROWS_PER_TILE, ROWS_REM)],
                        out_hbm.at[cid, pl.ds(NS * ROWS_PER_TILE,
                                              ROWS_REM)])


def _msg_agg(h, ee, src, dst, ew):
    mesh = plsc.VectorSubcoreMesh(core_axis_name="c", subcore_axis_name="s",
                                  num_cores=NC, num_subcores=NS)
    f = pl.kernel(
        _msg_agg_body,
        mesh=mesh,
        out_type=jax.ShapeDtypeStruct((NC, N, D), jnp.float32),
        scratch_types=(
            [pltpu.VMEM((C,), jnp.int32),
             pltpu.VMEM((C,), jnp.int32),
             pltpu.VMEM((C, 16), jnp.float32),
             pltpu.VMEM((C, D), jnp.float32),
             pltpu.VMEM((C, D), jnp.float32)] * 2
            + [pltpu.VMEM((ZROWS, D), jnp.float32),
               pltpu.VMEM_SHARED((N, D), jnp.float32)]
            + [pltpu.SemaphoreType.DMA] * 8),
    )
    return f(h, ee, src, dst, ew)


def _node_update_body(h_ref, parts_ref, eps_ref, w1_ref, b1_ref, w2_ref,
                      b2_ref, gamma_ref, beta_ref, out_ref):
    agg = parts_ref[0] + parts_ref[1]
    z = (1.0 + eps_ref[0, 0]) * h_ref[...] + agg
    z = jnp.maximum(
        jnp.dot(z, w1_ref[...], preferred_element_type=jnp.float32)
        + b1_ref[...], 0.0)
    z = (jnp.dot(z, w2_ref[...], preferred_element_type=jnp.float32)
         + b2_ref[...])
    mu = jnp.mean(z, axis=0, keepdims=True)
    var = jnp.mean((z - mu) ** 2, axis=0, keepdims=True)
    z = (z - mu) / jnp.sqrt(var + 1e-5) * gamma_ref[...] + beta_ref[...]
    out_ref[...] = jnp.maximum(z, 0.0)


def _node_update(h, parts, eps, w1, b1, w2, b2, gamma, beta):
    return pl.pallas_call(
        _node_update_body,
        out_shape=jax.ShapeDtypeStruct((N, D), jnp.float32),
    )(h, parts, eps, w1, b1, w2, b2, gamma, beta)


def _pool_head_body(h_ref, batch_ref, w1_ref, b1_ref, w2_ref, b2_ref,
                    w3_ref, b3_ref, w4_ref, b4_ref, out_ref):
    gids = lax.broadcasted_iota(jnp.int32, (N, G), 1)
    oh = (batch_ref[...] == gids).astype(jnp.float32)
    cnt = jnp.sum(oh, axis=0, keepdims=True)
    pooled = jax.lax.dot_general(
        oh, h_ref[...], (((0,), (0,)), ((), ())),
        preferred_element_type=jnp.float32)
    pooled = pooled / jnp.maximum(cnt, 1.0).reshape(G, 1)
    g = jnp.maximum(
        jnp.dot(pooled, w1_ref[...], preferred_element_type=jnp.float32)
        + b1_ref[...], 0.0)
    g = jnp.maximum(
        jnp.dot(g, w2_ref[...], preferred_element_type=jnp.float32)
        + b2_ref[...], 0.0)
    g = jnp.maximum(
        jnp.dot(g, w3_ref[...], preferred_element_type=jnp.float32)
        + b3_ref[...], 0.0)
    out_ref[...] = (
        jnp.dot(g, w4_ref[...], preferred_element_type=jnp.float32)
        + b4_ref[...])


def _pool_head(h, batch2d, p):
    return pl.pallas_call(
        _pool_head_body,
        out_shape=jax.ShapeDtypeStruct((G, 1), jnp.float32),
    )(h, batch2d,
      p["fcW_1"], p["fcb_1"].reshape(1, D),
      p["fcW_2"], p["fcb_2"].reshape(1, D),
      p["fcW_3"], p["fcb_3"].reshape(1, D),
      p["fcW_4"], p["fcb_4"].reshape(1, 1))


def kernel(x, edge_index, edge_attr, edge_weight, batch, params):
    src = edge_index[0]
    dst = edge_index[1]
    L = 3
    ee0, ewb = _edge_mlp1(edge_attr, edge_weight.reshape(E, 1),
                          params["Wb1_0"], params["bb1_0"].reshape(1, D),
                          params["Wb2_0"], params["bb2_0"].reshape(1, D))
    w1s = jnp.stack([params[f"Wb1_{l}"] for l in (1, 2)])
    b1s = jnp.stack([params[f"bb1_{l}"].reshape(1, D) for l in (1, 2)])
    w2s = jnp.stack([params[f"Wb2_{l}"] for l in (1, 2)])
    b2s = jnp.stack([params[f"bb2_{l}"].reshape(1, D) for l in (1, 2)])
    h = x
    parts0 = _msg_agg(h, ee0, src, dst, ewb)
    ee1, ee2 = _edge_mlp2(edge_attr, w1s, b1s, w2s, b2s)
    ee = (ee0, ee1, ee2)
    for l in range(L):
        parts = parts0 if l == 0 else _msg_agg(h, ee[l], src, dst, ewb)
        h = _node_update(
            h, parts, params[f"eps_{l}"].reshape(1, 1),
            params[f"Wm1_{l}"], params[f"bm1_{l}"].reshape(1, D),
            params[f"Wm2_{l}"], params[f"bm2_{l}"].reshape(1, D),
            params[f"gamma_{l}"].reshape(1, D),
            params[f"beta_{l}"].reshape(1, D))
    out = _pool_head(h, batch.reshape(N, 1), params)
    return out.reshape(-1)


# speedup vs baseline: 3.8898x; 1.2038x over previous
import functools

import jax
import jax.numpy as jnp
from jax import lax
from jax.experimental import pallas as pl
from jax.experimental.pallas import tpu as pltpu
from jax.experimental.pallas import tpu_sc as plsc

N = 10000
E = 320000
D = 128
DE = 16
G = 64

NC = 2
NS = 16
C = 64
E_PER_SC = E // NC
CHUNKS_PER_SC = E_PER_SC // C
ROWS_PER_TILE = 624
ROWS_REM = N - NS * ROWS_PER_TILE
ZROWS = 8


BE = 2000


def _edge_mlp1_body(ea_ref, ew_ref, w1_ref, b1_ref, w2_ref, b2_ref,
                    o0_ref, ewb_ref):
    t = jnp.maximum(
        jnp.dot(ea_ref[...], w1_ref[...],
                preferred_element_type=jnp.float32) + b1_ref[...], 0.0)
    o0_ref[...] = (
        jnp.dot(t, w2_ref[...], preferred_element_type=jnp.float32)
        + b2_ref[...])
    ewb_ref[...] = jnp.broadcast_to(ew_ref[...], (BE, 16))


def _edge_mlp1(edge_attr, ew2d, w1, b1, w2, b2):
    return pl.pallas_call(
        _edge_mlp1_body,
        grid=(E // BE,),
        in_specs=[
            pl.BlockSpec((BE, DE), lambda i: (i, 0)),
            pl.BlockSpec((BE, 1), lambda i: (i, 0)),
            pl.BlockSpec((DE, D), lambda i: (0, 0)),
            pl.BlockSpec((1, D), lambda i: (0, 0)),
            pl.BlockSpec((D, D), lambda i: (0, 0)),
            pl.BlockSpec((1, D), lambda i: (0, 0)),
        ],
        out_specs=[
            pl.BlockSpec((BE, D), lambda i: (i, 0)),
            pl.BlockSpec((BE, 16), lambda i: (i, 0)),
        ],
        out_shape=[
            jax.ShapeDtypeStruct((E, D), jnp.float32),
            jax.ShapeDtypeStruct((E, 16), jnp.float32),
        ],
    )(edge_attr, ew2d, w1, b1, w2, b2)


def _edge_mlp2_body(ea_ref, w1_ref, b1_ref, w2_ref, b2_ref,
                    o1_ref, o2_ref):
    ea = ea_ref[...]
    for l, o_ref in enumerate((o1_ref, o2_ref)):
        t = jnp.maximum(
            jnp.dot(ea, w1_ref[l], preferred_element_type=jnp.float32)
            + b1_ref[l], 0.0)
        o_ref[...] = (
            jnp.dot(t, w2_ref[l], preferred_element_type=jnp.float32)
            + b2_ref[l])


def _edge_mlp2(edge_attr, w1s, b1s, w2s, b2s):
    return pl.pallas_call(
        _edge_mlp2_body,
        grid=(E // BE,),
        in_specs=[
            pl.BlockSpec((BE, DE), lambda i: (i, 0)),
            pl.BlockSpec((2, DE, D), lambda i: (0, 0, 0)),
            pl.BlockSpec((2, 1, D), lambda i: (0, 0, 0)),
            pl.BlockSpec((2, D, D), lambda i: (0, 0, 0)),
            pl.BlockSpec((2, 1, D), lambda i: (0, 0, 0)),
        ],
        out_specs=[
            pl.BlockSpec((BE, D), lambda i: (i, 0)),
            pl.BlockSpec((BE, D), lambda i: (i, 0)),
        ],
        out_shape=[
            jax.ShapeDtypeStruct((E, D), jnp.float32),
            jax.ShapeDtypeStruct((E, D), jnp.float32),
        ],
    )(edge_attr, w1s, b1s, w2s, b2s)


def _msg_agg_body(h_hbm, ee_hbm, src_hbm, dst_hbm, ew_hbm, out_hbm,
                  src0, dst0, ew0, hr0, eev0,
                  src1, dst1, ew1, hr1, eev1,
                  zbuf_v, agg_sh, sem_i0, sem_i1, sem_g0, sem_g1,
                  sem_d0, sem_d1, sem_s0, sem_s1, sem_e0, sem_e1):
    bufs = ((src0, dst0, ew0, hr0, eev0, sem_i0, sem_g0, sem_d0, sem_s0,
             sem_e0),
            (src1, dst1, ew1, hr1, eev1, sem_i1, sem_g1, sem_d1, sem_s1,
             sem_e1))
    cid = lax.axis_index("c")
    sid = lax.axis_index("s")

    def zero_row(i, _):
        for d8 in range(D // 16):
            zbuf_v[i, pl.ds(d8 * 16, 16)] = jnp.zeros((16,), jnp.float32)
        return 0
    lax.fori_loop(0, ZROWS, zero_row, 0)
    rows_start = sid * ROWS_PER_TILE

    def zfill(k, _):
        pltpu.sync_copy(zbuf_v, agg_sh.at[pl.ds(rows_start + k * ZROWS,
                                                ZROWS)])
        return 0
    lax.fori_loop(0, ROWS_PER_TILE // ZROWS, zfill, 0)

    @pl.when(sid == NS - 1)
    def _():
        for k in range(ROWS_REM // ZROWS):
            pltpu.sync_copy(
                zbuf_v,
                agg_sh.at[pl.ds(NS * ROWS_PER_TILE + k * ZROWS, ZROWS)])
    plsc.subcore_barrier()

    n_g = (CHUNKS_PER_SC - sid + NS - 1) // NS
    e_base = cid * E_PER_SC

    def issue_idx(j, par):
        src_v, _, ew_v, _, ee_v = bufs[par][:5]
        sem_i, sem_e = bufs[par][5], bufs[par][9]
        off = e_base + (sid + j * NS) * C
        pltpu.async_copy(src_hbm.at[pl.ds(off, C)], src_v, sem_i)
        pltpu.async_copy(ew_hbm.at[pl.ds(off, C)], ew_v, sem_e)
        pltpu.async_copy(ee_hbm.at[pl.ds(off, C)], ee_v, sem_e)

    def wait_src(par):
        src_v, sem_i = bufs[par][0], bufs[par][5]
        pltpu.make_async_copy(src_hbm.at[pl.ds(0, C)], src_v, sem_i).wait()

    def wait_ee(par):
        ew_v, ee_v, sem_e = bufs[par][2], bufs[par][4], bufs[par][9]
        pltpu.make_async_copy(ew_hbm.at[pl.ds(0, C)], ew_v, sem_e).wait()
        pltpu.make_async_copy(ee_hbm.at[pl.ds(0, C)], ee_v, sem_e).wait()

    def issue_dst(j, par):
        dst_v, sem_d = bufs[par][1], bufs[par][7]
        off = e_base + (sid + j * NS) * C
        pltpu.async_copy(dst_hbm.at[pl.ds(off, C)], dst_v, sem_d)

    def wait_dst(par):
        dst_v, sem_d = bufs[par][1], bufs[par][7]
        pltpu.make_async_copy(dst_hbm.at[pl.ds(0, C)], dst_v, sem_d).wait()

    def issue_gather(par):
        src_v, hr_v, sem_g = bufs[par][0], bufs[par][3], bufs[par][6]
        pltpu.async_copy(h_hbm.at[src_v], hr_v, sem_g)

    def wait_gather(par):
        src_v, hr_v, sem_g = bufs[par][0], bufs[par][3], bufs[par][6]
        pltpu.make_async_copy(h_hbm.at[src_v], hr_v, sem_g).wait()

    def issue_scatter(par):
        dst_v, hr_v, sem_s = bufs[par][1], bufs[par][3], bufs[par][8]
        pltpu.async_copy(hr_v, agg_sh.at[dst_v], sem_s, add=True)

    def wait_scatter(par):
        dst_v, hr_v, sem_s = bufs[par][1], bufs[par][3], bufs[par][8]
        pltpu.make_async_copy(hr_v, agg_sh.at[dst_v], sem_s).wait()

    issue_idx(0, 0)
    issue_dst(0, 0)
    wait_src(0)
    issue_gather(0)
    issue_idx(1, 1)

    def pair_body(jj, _):
        for par in range(2):
            j = jj * 2 + par
            nb = 1 - par

            @pl.when(j < n_g)
            def _():
                wait_gather(par)

                @pl.when(j + 1 < n_g)
                def _():
                    wait_src(nb)

                    @pl.when(j >= 1)
                    def _():
                        wait_scatter(nb)
                    issue_gather(nb)
                    issue_dst(j + 1, nb)

                wait_ee(par)
                src_v, dst_v, ew_v, hr_v, ee_v = bufs[par][:5]

                def edge_body(c, _):
                    w = ew_v[c, pl.ds(0, 16)]
                    for d8 in range(D // 16):
                        sl = pl.ds(d8 * 16, 16)
                        hr_v[c, sl] = jnp.maximum(
                            hr_v[c, sl] + ee_v[c, sl], 0.0) * w
                    return 0
                lax.fori_loop(0, C, edge_body, 0)

                wait_dst(par)
                issue_scatter(par)

                @pl.when(j + 2 < n_g)
                def _():
                    issue_idx(j + 2, par)
        return 0
    lax.fori_loop(0, (n_g + 1) // 2, pair_body, 0)
    wait_scatter(0)
    wait_scatter(1)

    plsc.subcore_barrier()
    pltpu.sync_copy(agg_sh.at[pl.ds(rows_start, ROWS_PER_TILE)],
                    out_hbm.at[cid, pl.ds(rows_start, ROWS_PER_TILE)])

    @pl.when(sid == NS - 1)
    def _():
        pltpu.sync_copy(agg_sh.at[pl.ds(NS * ROWS_PER_TILE, ROWS_REM)],
                        out_hbm.at[cid, pl.ds(NS * ROWS_PER_TILE,
                                              ROWS_REM)])


def _msg_agg(h, ee, src, dst, ew):
    mesh = plsc.VectorSubcoreMesh(core_axis_name="c", subcore_axis_name="s",
                                  num_cores=NC, num_subcores=NS)
    f = pl.kernel(
        _msg_agg_body,
        mesh=mesh,
        out_type=jax.ShapeDtypeStruct((NC, N, D), jnp.float32),
        scratch_types=(
            [pltpu.VMEM((C,), jnp.int32),
             pltpu.VMEM((C,), jnp.int32),
             pltpu.VMEM((C, 16), jnp.float32),
             pltpu.VMEM((C, D), jnp.float32),
             pltpu.VMEM((C, D), jnp.float32)] * 2
            + [pltpu.VMEM((ZROWS, D), jnp.float32),
               pltpu.VMEM_SHARED((N, D), jnp.float32)]
            + [pltpu.SemaphoreType.DMA] * 10),
    )
    return f(h, ee, src, dst, ew)


def _node_update_body(h_ref, parts_ref, eps_ref, w1_ref, b1_ref, w2_ref,
                      b2_ref, gamma_ref, beta_ref, out_ref):
    agg = parts_ref[0] + parts_ref[1]
    z = (1.0 + eps_ref[0, 0]) * h_ref[...] + agg
    z = jnp.maximum(
        jnp.dot(z, w1_ref[...], preferred_element_type=jnp.float32)
        + b1_ref[...], 0.0)
    z = (jnp.dot(z, w2_ref[...], preferred_element_type=jnp.float32)
         + b2_ref[...])
    mu = jnp.mean(z, axis=0, keepdims=True)
    var = jnp.mean((z - mu) ** 2, axis=0, keepdims=True)
    z = (z - mu) / jnp.sqrt(var + 1e-5) * gamma_ref[...] + beta_ref[...]
    out_ref[...] = jnp.maximum(z, 0.0)


def _node_update(h, parts, eps, w1, b1, w2, b2, gamma, beta):
    return pl.pallas_call(
        _node_update_body,
        out_shape=jax.ShapeDtypeStruct((N, D), jnp.float32),
    )(h, parts, eps, w1, b1, w2, b2, gamma, beta)


def _pool_head_body(h_ref, batch_ref, w1_ref, b1_ref, w2_ref, b2_ref,
                    w3_ref, b3_ref, w4_ref, b4_ref, out_ref):
    gids = lax.broadcasted_iota(jnp.int32, (N, G), 1)
    oh = (batch_ref[...] == gids).astype(jnp.float32)
    cnt = jnp.sum(oh, axis=0, keepdims=True)
    pooled = jax.lax.dot_general(
        oh, h_ref[...], (((0,), (0,)), ((), ())),
        preferred_element_type=jnp.float32)
    pooled = pooled / jnp.maximum(cnt, 1.0).reshape(G, 1)
    g = jnp.maximum(
        jnp.dot(pooled, w1_ref[...], preferred_element_type=jnp.float32)
        + b1_ref[...], 0.0)
    g = jnp.maximum(
        jnp.dot(g, w2_ref[...], preferred_element_type=jnp.float32)
        + b2_ref[...], 0.0)
    g = jnp.maximum(
        jnp.dot(g, w3_ref[...], preferred_element_type=jnp.float32)
        + b3_ref[...], 0.0)
    out_ref[...] = (
        jnp.dot(g, w4_ref[...], preferred_element_type=jnp.float32)
        + b4_ref[...])


def _pool_head(h, batch2d, p):
    return pl.pallas_call(
        _pool_head_body,
        out_shape=jax.ShapeDtypeStruct((G, 1), jnp.float32),
    )(h, batch2d,
      p["fcW_1"], p["fcb_1"].reshape(1, D),
      p["fcW_2"], p["fcb_2"].reshape(1, D),
      p["fcW_3"], p["fcb_3"].reshape(1, D),
      p["fcW_4"], p["fcb_4"].reshape(1, 1))


def kernel(x, edge_index, edge_attr, edge_weight, batch, params):
    src = edge_index[0]
    dst = edge_index[1]
    L = 3
    ee0, ewb = _edge_mlp1(edge_attr, edge_weight.reshape(E, 1),
                          params["Wb1_0"], params["bb1_0"].reshape(1, D),
                          params["Wb2_0"], params["bb2_0"].reshape(1, D))
    w1s = jnp.stack([params[f"Wb1_{l}"] for l in (1, 2)])
    b1s = jnp.stack([params[f"bb1_{l}"].reshape(1, D) for l in (1, 2)])
    w2s = jnp.stack([params[f"Wb2_{l}"] for l in (1, 2)])
    b2s = jnp.stack([params[f"bb2_{l}"].reshape(1, D) for l in (1, 2)])
    h = x
    parts0 = _msg_agg(h, ee0, src, dst, ewb)
    ee1, ee2 = _edge_mlp2(edge_attr, w1s, b1s, w2s, b2s)
    ee = (ee0, ee1, ee2)
    for l in range(L):
        parts = parts0 if l == 0 else _msg_agg(h, ee[l], src, dst, ewb)
        h = _node_update(
            h, parts, params[f"eps_{l}"].reshape(1, 1),
            params[f"Wm1_{l}"], params[f"bm1_{l}"].reshape(1, D),
            params[f"Wm2_{l}"], params[f"bm2_{l}"].reshape(1, D),
            params[f"gamma_{l}"].reshape(1, D),
            params[f"beta_{l}"].reshape(1, D))
    out = _pool_head(h, batch.reshape(N, 1), params)
    return out.reshape(-1)
